# baseline (speedup 1.0000x reference)
import functools

import jax
import jax.numpy as jnp
from jax import lax
from jax.experimental import pallas as pl
from jax.experimental.pallas import tpu as pltpu
from jax.experimental.pallas import tpu_sc as plsc

N = 8 * 224 * 224
C = 96
OC = 48
NW = 32
RPW = N // NW
CR = 224
NITER = RPW // CR

_mesh = plsc.VectorSubcoreMesh(core_axis_name="c", subcore_axis_name="s")


@functools.partial(
    pl.kernel,
    mesh=_mesh,
    out_type=jax.ShapeDtypeStruct((N, OC), jnp.float32),
    scratch_types=[
        pltpu.VMEM((CR, C), jnp.float32),
        pltpu.VMEM((CR, C), jnp.float32),
        pltpu.VMEM((CR, OC), jnp.float32),
        pltpu.VMEM((CR, OC), jnp.float32),
        pltpu.SemaphoreType.DMA,
        pltpu.SemaphoreType.DMA,
        pltpu.SemaphoreType.DMA,
        pltpu.SemaphoreType.DMA,
    ],
    compiler_params=pltpu.CompilerParams(needs_layout_passes=False),
)
def _sel(in_hbm, out_hbm, in0, in1, out0, out1, si0, si1, so0, so1):
    wid = lax.axis_index("s") * 2 + lax.axis_index("c")
    base = wid * RPW
    lanes = lax.iota(jnp.int32, 16)
    lanes2 = lanes * 2

    def in_cp(i, buf, sem):
        row = pl.multiple_of(base + i * CR, 8)
        return pltpu.make_async_copy(in_hbm.at[pl.ds(row, CR), :], buf, sem)

    def out_cp(i, buf, sem):
        row = pl.multiple_of(base + i * CR, 8)
        return pltpu.make_async_copy(buf, out_hbm.at[pl.ds(row, CR), :], sem)

    def compute(src, dst):
        @plsc.parallel_loop(0, CR, unroll=4)
        def _(r):
            rows = jnp.full((16,), r, jnp.int32)
            for k in range(OC // 16):
                dst[r, pl.ds(k * 16, 16)] = plsc.load_gather(
                    src, [rows, k * 32 + lanes2]
                )

    in_cp(0, in0, si0).start()

    def body(g, carry):
        i0 = g * 2
        i1 = i0 + 1

        in_cp(i0, in0, si0).wait()
        in_cp(i1, in1, si1).start()

        @pl.when(g > 0)
        def _():
            out_cp(i0 - 2, out0, so0).wait()

        compute(in0, out0)
        out_cp(i0, out0, so0).start()

        in_cp(i1, in1, si1).wait()

        @pl.when(g < NITER // 2 - 1)
        def _():
            in_cp(i0 + 2, in0, si0).start()

        @pl.when(g > 0)
        def _():
            out_cp(i1 - 2, out1, so1).wait()

        compute(in1, out1)
        out_cp(i1, out1, so1).start()
        return carry

    lax.fori_loop(0, NITER // 2, body, 0)
    out_cp(NITER - 2, out0, so0).wait()
    out_cp(NITER - 1, out1, so1).wait()


def kernel(inputs):
    mat = inputs.reshape(N, C)
    out = _sel(mat)
    return out.reshape(8, 224, 224, 48)
